# baseline (speedup 1.0000x reference)
import functools

import jax
import jax.numpy as jnp
from jax.experimental import pallas as pl
from jax.experimental.pallas import tpu as pltpu

N_SIDE = 10000
E = 320000
IN_DIM = 128
HID = 128
NH = 4
N_TOT = 2 * N_SIDE
NPG = N_SIDE // 8

_ROWS = 200


def _align_body(q_ref, kt_ref, v_ref, o_ref):
    att = jnp.dot(q_ref[...], kt_ref[...], preferred_element_type=jnp.float32)
    m = jnp.max(att, axis=-1, keepdims=True)
    e = jnp.exp(att - m)
    s = jnp.sum(e, axis=-1, keepdims=True)
    o_ref[...] = jnp.dot(e / s, v_ref[...], preferred_element_type=jnp.float32)


@jax.jit
def _align(q, kt, v):
    return pl.pallas_call(
        _align_body,
        grid=(N_SIDE // _ROWS,),
        in_specs=[
            pl.BlockSpec((_ROWS, IN_DIM), lambda i: (i, 0)),
            pl.BlockSpec((IN_DIM, N_SIDE), lambda i: (0, 0)),
            pl.BlockSpec((N_SIDE, IN_DIM), lambda i: (0, 0)),
        ],
        out_specs=pl.BlockSpec((_ROWS, IN_DIM), lambda i: (i, 0)),
        out_shape=jax.ShapeDtypeStruct((N_SIDE, IN_DIM), jnp.float32),
    )(q, kt, v)


def _leaky(x):
    return jnp.where(x >= 0, x, 0.2 * x)


def _gat(h, src, dst, W, al, ar):
    N = h.shape[0]
    ft = (h @ W).reshape(N, NH, HID)
    a1 = jnp.einsum('nhd,hdo->nho', ft, al)
    a2 = jnp.einsum('nhd,hdo->nho', ft, ar)
    e = _leaky(a1[src] + a2[dst])
    emax = jax.ops.segment_max(e, dst, num_segments=N)
    emax = jnp.where(jnp.isfinite(emax), emax, 0.0)
    ee = jnp.exp(e - emax[dst])
    esum = jax.ops.segment_sum(ee, dst, num_segments=N)
    a = ee / (esum[dst] + 1e-9)
    out = jax.ops.segment_sum(ft[src] * a, dst, num_segments=N)
    return jax.nn.elu(out.reshape(N, -1))


def _bn_eval(x, g, b):
    return x / jnp.sqrt(1.0 + 1e-5) * g + b


def kernel(x1, x2, edge_index, edge_embeddings, W1, attn_l1, attn_r1, W2,
           attn_l2, attn_r2, bn1_g, bn1_b, lin1_W, lin1_b, bn2_g, bn2_b,
           lin2_W, lin2_b):
    src = edge_index[0]
    dst = edge_index[1]
    x1a = _align(x1, x2.T, x2)
    x2a = _align(x2, x1.T, x1)
    q1 = jnp.concatenate([x1, x1a, x1 - x1a, x1 * x1a], axis=-1)
    q2 = jnp.concatenate([x2, x2a, x2 - x2a, x2 * x2a], axis=-1)
    h = jnp.concatenate([q1, q2], axis=0)
    h = _gat(h, src, dst, W1, attn_l1, attn_r1)
    h = _gat(h, src, dst, W2, attn_l2, attn_r2)
    hcat = jnp.concatenate([h, edge_embeddings], axis=1)
    gid = jnp.arange(N_TOT) // NPG
    hg = jax.ops.segment_sum(hcat, gid, num_segments=16) / float(NPG)
    xcls = jnp.concatenate([hg[:8], hg[8:]], axis=1)
    y = _bn_eval(xcls, bn1_g, bn1_b)
    y = y @ lin1_W + lin1_b
    y = jax.nn.relu(y)
    y = _bn_eval(y, bn2_g, bn2_b)
    return y @ lin2_W + lin2_b


# speedup vs baseline: 3.5166x; 3.5166x over previous
import functools

import jax
import jax.numpy as jnp
from jax.experimental import pallas as pl
from jax.experimental.pallas import tpu as pltpu
from jax.experimental.pallas import tpu_sc as plsc

N_SIDE = 10000
E = 320000
IN_DIM = 128
HID = 128
NH = 4
N_TOT = 2 * N_SIDE
NPG = N_SIDE // 8

_ROWS = 200


def _align_body(q_ref, kt_ref, v_ref, o_ref):
    att = jnp.dot(q_ref[...], kt_ref[...], preferred_element_type=jnp.float32)
    m = jnp.max(att, axis=-1, keepdims=True)
    e = jnp.exp(att - m)
    s = jnp.sum(e, axis=-1, keepdims=True)
    o_ref[...] = jnp.dot(e / s, v_ref[...], preferred_element_type=jnp.float32)


@jax.jit
def _align(q, kt, v):
    return pl.pallas_call(
        _align_body,
        grid=(N_SIDE // _ROWS,),
        in_specs=[
            pl.BlockSpec((_ROWS, IN_DIM), lambda i: (i, 0)),
            pl.BlockSpec((IN_DIM, N_SIDE), lambda i: (0, 0)),
            pl.BlockSpec((N_SIDE, IN_DIM), lambda i: (0, 0)),
        ],
        out_specs=pl.BlockSpec((_ROWS, IN_DIM), lambda i: (i, 0)),
        out_shape=jax.ShapeDtypeStruct((N_SIDE, IN_DIM), jnp.float32),
    )(q, kt, v)


def _leaky(x):
    return jnp.where(x >= 0, x, 0.2 * x)


_NW = 32
_AROWS = 128
_NROUND = 5
_SBLK = 2000
_NB = E // _SBLK
_LCAP = 112
_FIRE = _LCAP - 16
_FDIM = NH * HID


def _agg_body(vals_hbm, dst_hbm, zeros_hbm, out_hbm,
              dbufA, dbufB, liste, listr, rows_v, acc, nref,
              semA, semB, gsem):
    c = jax.lax.axis_index("c")
    s = jax.lax.axis_index("s")
    wid = c * 16 + s
    iota16 = jax.lax.iota(jnp.int32, 16)
    zero16 = jnp.zeros((16,), jnp.int32)

    def clear_list():
        for q in range(_LCAP // 16):
            liste[pl.ds(16 * q, 16)] = zero16

    def fire():
        nn = nref[0]
        pltpu.async_copy(vals_hbm.at[liste], rows_v, gsem).wait()

        def acc_edge(i, carry):
            r = listr[pl.ds(i, 16)][0]
            for v in range(_FDIM // 16):
                plsc.addupdate(acc.at[r, pl.ds(16 * v, 16)],
                               rows_v[i, pl.ds(16 * v, 16)])
            return carry

        jax.lax.fori_loop(0, nn, acc_edge, jnp.int32(0))
        clear_list()
        nref[0] = jnp.int32(0)

    for R in range(_NROUND):
        lo = (R * _NW + wid) * _AROWS
        hi = jnp.minimum(lo + _AROWS, N_TOT)
        pltpu.sync_copy(zeros_hbm, acc)
        clear_list()
        nref[0] = jnp.int32(0)

        def scan(j, buf, carry):
            base = j * _SBLK

            def vec(i, carry2):
                off = i * 16
                d16 = buf[pl.ds(off, 16)]
                inb = (d16 >= lo) & (d16 < hi)
                n0 = nref[0]
                plsc.store_compressed(liste.at[pl.ds(n0, 16)],
                                      base + off + iota16, mask=inb)
                plsc.store_compressed(listr.at[pl.ds(n0, 16)],
                                      d16 - lo, mask=inb)
                n1 = n0 + jnp.sum(inb.astype(jnp.int32))
                nref[0] = n1

                @pl.when(n1 > _FIRE)
                def _():
                    fire()
                return carry2

            return jax.lax.fori_loop(0, _SBLK // 16, vec, carry)

        pltpu.async_copy(dst_hbm.at[pl.ds(0, _SBLK)], dbufA, semA)

        def pair(p, carry):
            j0 = 2 * p
            pltpu.make_async_copy(
                dst_hbm.at[pl.ds(j0 * _SBLK, _SBLK)], dbufA, semA).wait()
            pltpu.async_copy(
                dst_hbm.at[pl.ds((j0 + 1) * _SBLK, _SBLK)], dbufB, semB)
            carry = scan(j0, dbufA, carry)

            @pl.when(p < _NB // 2 - 1)
            def _():
                pltpu.async_copy(
                    dst_hbm.at[pl.ds((j0 + 2) * _SBLK, _SBLK)], dbufA, semA)
            pltpu.make_async_copy(
                dst_hbm.at[pl.ds((j0 + 1) * _SBLK, _SBLK)], dbufB, semB).wait()
            carry = scan(j0 + 1, dbufB, carry)
            return carry

        jax.lax.fori_loop(0, _NB // 2, pair, jnp.int32(0))

        @pl.when(nref[0] > 0)
        def _():
            fire()

        @pl.when(lo + _AROWS <= N_TOT)
        def _():
            pltpu.sync_copy(acc, out_hbm.at[pl.ds(lo, _AROWS)])

        @pl.when((lo < N_TOT) & (lo + _AROWS > N_TOT))
        def _():
            pltpu.sync_copy(acc.at[pl.ds(0, N_TOT % _AROWS)],
                            out_hbm.at[pl.ds(lo, N_TOT % _AROWS)])


_sc_params = pltpu.CompilerParams()
if "needs_layout_passes" in pltpu.CompilerParams.__dataclass_fields__:
    import dataclasses as _dc
    _sc_params = _dc.replace(_sc_params, needs_layout_passes=False)

_scatter_agg = pl.kernel(
    _agg_body,
    out_type=jax.ShapeDtypeStruct((N_TOT, _FDIM), jnp.float32),
    mesh=plsc.VectorSubcoreMesh(core_axis_name="c", subcore_axis_name="s"),
    compiler_params=_sc_params,
    scratch_types=[
        pltpu.VMEM((_SBLK,), jnp.int32),
        pltpu.VMEM((_SBLK,), jnp.int32),
        pltpu.VMEM((_LCAP,), jnp.int32),
        pltpu.VMEM((_LCAP + 16,), jnp.int32),
        pltpu.VMEM((_LCAP, _FDIM), jnp.float32),
        pltpu.VMEM((_AROWS, _FDIM), jnp.float32),
        pltpu.SMEM((1,), jnp.int32),
        pltpu.SemaphoreType.DMA,
        pltpu.SemaphoreType.DMA,
        pltpu.SemaphoreType.DMA,
    ],
)


def _gat(h, src, dst, W, al, ar, zeros):
    N = h.shape[0]
    ft = (h @ W).reshape(N, NH, HID)
    a1 = jnp.einsum('nhd,hdo->nho', ft, al)
    a2 = jnp.einsum('nhd,hdo->nho', ft, ar)
    e = _leaky(a1[src] + a2[dst])
    emax = jax.ops.segment_max(e, dst, num_segments=N)
    emax = jnp.where(jnp.isfinite(emax), emax, 0.0)
    ee = jnp.exp(e - emax[dst])
    esum = jax.ops.segment_sum(ee, dst, num_segments=N)
    a = ee / (esum[dst] + 1e-9)
    vals = (ft[src] * a).reshape(E, _FDIM)
    out = _scatter_agg(vals, dst, zeros)
    return jax.nn.elu(out)


def _bn_eval(x, g, b):
    return x / jnp.sqrt(1.0 + 1e-5) * g + b


def kernel(x1, x2, edge_index, edge_embeddings, W1, attn_l1, attn_r1, W2,
           attn_l2, attn_r2, bn1_g, bn1_b, lin1_W, lin1_b, bn2_g, bn2_b,
           lin2_W, lin2_b):
    src = edge_index[0].astype(jnp.int32)
    dst = edge_index[1].astype(jnp.int32)
    zeros = jnp.zeros((_AROWS, _FDIM), jnp.float32)
    x1a = _align(x1, x2.T, x2)
    x2a = _align(x2, x1.T, x1)
    q1 = jnp.concatenate([x1, x1a, x1 - x1a, x1 * x1a], axis=-1)
    q2 = jnp.concatenate([x2, x2a, x2 - x2a, x2 * x2a], axis=-1)
    h = jnp.concatenate([q1, q2], axis=0)
    h = _gat(h, src, dst, W1, attn_l1, attn_r1, zeros)
    h = _gat(h, src, dst, W2, attn_l2, attn_r2, zeros)
    hcat = jnp.concatenate([h, edge_embeddings], axis=1)
    gid = jnp.arange(N_TOT) // NPG
    hg = jax.ops.segment_sum(hcat, gid, num_segments=16) / float(NPG)
    xcls = jnp.concatenate([hg[:8], hg[8:]], axis=1)
    y = _bn_eval(xcls, bn1_g, bn1_b)
    y = y @ lin1_W + lin1_b
    y = jax.nn.relu(y)
    y = _bn_eval(y, bn2_g, bn2_b)
    return y @ lin2_W + lin2_b
